# baseline (speedup 1.0000x reference)
import functools

import jax
import jax.numpy as jnp
from jax import lax
from jax.experimental import pallas as pl
from jax.experimental.pallas import tpu as pltpu
from jax.experimental.pallas import tpu_sc as plsc

EMBED_DIM = 32

_NC = 2
_NS = 16
_NW = _NC * _NS

_TOT = 16384 * 20
_PER_W = _TOT // _NW
_CHUNK = 512
_NCHUNK = _PER_W // _CHUNK
_NBUF = 4

_mesh = plsc.VectorSubcoreMesh(core_axis_name="c", subcore_axis_name="s")

_VOCAB = 1_000_000
_ROWS = 256
_NBLK = _VOCAB // _ROWS
_UTAIL = _VOCAB - _NBLK * _ROWS
_UBLK_PER_W = _NBLK // _NW
_UEXTRA_W = _NBLK - _UBLK_PER_W * _NW


@functools.partial(
    pl.kernel,
    mesh=_mesh,
    out_type=jax.ShapeDtypeStruct((_VOCAB * EMBED_DIM,), jnp.float32),
    scratch_types=(
        [pltpu.VMEM((_ROWS, EMBED_DIM), jnp.float32) for _ in range(2)]
        + [pltpu.VMEM((_ROWS * EMBED_DIM,), jnp.float32) for _ in range(2)]
        + [pltpu.SemaphoreType.DMA for _ in range(4)]
    ),
    compiler_params=pltpu.CompilerParams(needs_layout_passes=False),
)
def _untile_kernel(tab, out_hbm, st0, st1, ov0, ov1, gi0, gi1, go0, go1):
    wid = lax.axis_index("s") * _NC + lax.axis_index("c")
    stages = (st0, st1)
    outs = (ov0, ov1)
    gsems = (gi0, gi1)
    wsems = (go0, go1)

    def t_of(j):
        return wid + _NW * j

    def start_in(t, b):
        pltpu.async_copy(tab.at[pl.ds(t * _ROWS, _ROWS)], stages[b], gsems[b])

    def wait_in(b):
        pltpu.make_async_copy(tab.at[pl.ds(0, _ROWS)], stages[b],
                              gsems[b]).wait()

    def wait_out(b):
        pltpu.make_async_copy(outs[b], out_hbm.at[pl.ds(0, _ROWS * 32)],
                              wsems[b]).wait()

    def compact(st, ov, nrows):
        def body(i, carry):
            for u in range(8):
                r = i * 8 + u
                ov[pl.ds(r * 32, 16)] = st[r, pl.ds(0, 16)]
                ov[pl.ds(r * 32 + 16, 16)] = st[r, pl.ds(16, 16)]
            return carry
        lax.fori_loop(0, nrows // 8, body, 0)

    start_in(t_of(0), 0)
    start_in(t_of(1), 1)
    for b in range(2):
        wait_in(b)
        compact(stages[b], outs[b], _ROWS)
        pltpu.async_copy(outs[b],
                         out_hbm.at[pl.ds(t_of(b) * _ROWS * 32, _ROWS * 32)],
                         wsems[b])
        start_in(jnp.minimum(t_of(b + 2), _NBLK - 1), b)

    def body(j2, carry):
        for b in range(2):
            j = j2 * 2 + b
            wait_in(b)
            wait_out(b)
            compact(stages[b], outs[b], _ROWS)
            pltpu.async_copy(
                outs[b],
                out_hbm.at[pl.ds(t_of(j) * _ROWS * 32, _ROWS * 32)],
                wsems[b])
            start_in(jnp.minimum(t_of(j + 2), _NBLK - 1), b)
        return carry

    lax.fori_loop(1, _UBLK_PER_W // 2, body, 0)

    for b in range(2):
        wait_out(b)
        wait_in(b)

    @pl.when(wid < _UEXTRA_W)
    def _():
        t = wid + _NW * _UBLK_PER_W
        pltpu.sync_copy(tab.at[pl.ds(t * _ROWS, _ROWS)], st0)
        compact(st0, ov0, _ROWS)
        pltpu.sync_copy(ov0, out_hbm.at[pl.ds(t * _ROWS * 32, _ROWS * 32)])

    @pl.when(wid == _NW - 1)
    def _():
        base = _NBLK * _ROWS
        pltpu.sync_copy(tab.at[pl.ds(base, _UTAIL)],
                        st1.at[pl.ds(0, _UTAIL)])
        compact(st1, ov1, _UTAIL)
        pltpu.sync_copy(ov1.at[pl.ds(0, _UTAIL * 32)],
                        out_hbm.at[pl.ds(base * 32, _UTAIL * 32)])


@functools.partial(
    pl.kernel,
    mesh=_mesh,
    out_type=jax.ShapeDtypeStruct((_TOT, EMBED_DIM), jnp.float32),
    scratch_types=(
        [pltpu.VMEM((_PER_W,), jnp.int32)]
        + [pltpu.VMEM((_CHUNK, EMBED_DIM), jnp.float32) for _ in range(_NBUF)]
        + [pltpu.SemaphoreType.DMA for _ in range(2 * _NBUF)]
    ),
    compiler_params=pltpu.CompilerParams(use_tc_tiling_on_sc=False),
)
def _gather_kernel(ids_hbm, table_hbm, out_hbm, idx_v, *bufs_sems):
    rows = bufs_sems[:_NBUF]
    gsem = bufs_sems[_NBUF:2 * _NBUF]
    wsem = bufs_sems[2 * _NBUF:]

    wid = lax.axis_index("s") * _NC + lax.axis_index("c")
    base = wid * _PER_W

    pltpu.sync_copy(ids_hbm.at[pl.ds(base, _PER_W)], idx_v)

    def start_gather(i, b):
        return pltpu.async_copy(
            table_hbm.at[idx_v.at[pl.ds(i * _CHUNK, _CHUNK)]], rows[b], gsem[b])

    def start_write(i, b):
        return pltpu.async_copy(
            rows[b], out_hbm.at[pl.ds(base + i * _CHUNK, _CHUNK)], wsem[b])

    g = [None] * _NBUF
    w = [None] * _NBUF
    for i in range(min(_NBUF, _NCHUNK)):
        g[i] = start_gather(i, i)
    for i in range(_NCHUNK):
        b = i % _NBUF
        g[b].wait()
        w[b] = start_write(i, b)
        j = i + _NBUF
        if j < _NCHUNK:
            w[b].wait()
            g[b] = start_gather(j, b)
        else:
            w[b].wait()


def kernel(input_ids, embedding_table):
    flat = input_ids.reshape(-1).astype(jnp.int32)
    lin = _untile_kernel(embedding_table)
    table_lin = lin.reshape(_VOCAB, EMBED_DIM)
    out = _gather_kernel(flat, table_lin)
    out = out.reshape(input_ids.shape + (EMBED_DIM,))
    return (out, embedding_table)


# speedup vs baseline: 1.0681x; 1.0681x over previous
import functools

import jax
import jax.numpy as jnp
from jax import lax
from jax.experimental import pallas as pl
from jax.experimental.pallas import tpu as pltpu
from jax.experimental.pallas import tpu_sc as plsc

EMBED_DIM = 32

_NC = 2
_NS = 16
_NW = _NC * _NS

_TOT = 16384 * 20
_PER_W = _TOT // _NW
_CHUNK = 512
_NCHUNK = _PER_W // _CHUNK
_NBUF = 4

_mesh = plsc.VectorSubcoreMesh(core_axis_name="c", subcore_axis_name="s")


@functools.partial(
    pl.kernel,
    mesh=_mesh,
    out_type=jax.ShapeDtypeStruct((_TOT, EMBED_DIM), jnp.float32),
    scratch_types=(
        [pltpu.VMEM((_PER_W,), jnp.int32)]
        + [pltpu.VMEM((_CHUNK, EMBED_DIM), jnp.float32) for _ in range(_NBUF)]
        + [pltpu.SemaphoreType.DMA for _ in range(2 * _NBUF)]
    ),
    compiler_params=pltpu.CompilerParams(use_tc_tiling_on_sc=False),
)
def _gather_kernel(ids_hbm, table_hbm, out_hbm, idx_v, *bufs_sems):
    rows = bufs_sems[:_NBUF]
    gsem = bufs_sems[_NBUF:2 * _NBUF]
    wsem = bufs_sems[2 * _NBUF:]

    wid = lax.axis_index("s") * _NC + lax.axis_index("c")
    base = wid * _PER_W

    pltpu.sync_copy(ids_hbm.at[pl.ds(base, _PER_W)], idx_v)

    def start_gather(i, b):
        return pltpu.async_copy(
            table_hbm.at[idx_v.at[pl.ds(i * _CHUNK, _CHUNK)]], rows[b], gsem[b])

    def start_write(i, b):
        return pltpu.async_copy(
            rows[b], out_hbm.at[pl.ds(base + i * _CHUNK, _CHUNK)], wsem[b])

    g = [None] * _NBUF
    w = [None] * _NBUF
    for i in range(min(_NBUF, _NCHUNK)):
        g[i] = start_gather(i, i)
    for i in range(_NCHUNK):
        b = i % _NBUF
        g[b].wait()
        w[b] = start_write(i, b)
        j = i + _NBUF
        if j < _NCHUNK:
            w[b].wait()
            g[b] = start_gather(j, b)
        else:
            w[b].wait()


def kernel(input_ids, embedding_table):
    flat = input_ids.reshape(-1).astype(jnp.int32)
    out = _gather_kernel(flat, embedding_table)
    out = out.reshape(input_ids.shape + (EMBED_DIM,))
    return (out, embedding_table)
